# baseline (speedup 1.0000x reference)
import functools

import jax
import jax.numpy as jnp
from jax import lax
from jax.experimental import pallas as pl
from jax.experimental.pallas import tpu as pltpu
from jax.experimental.pallas import tpu_sc as plsc

NUM_PHONEMES = 80
NUM_STRESS = 4
HIDDEN = 256

NC = 2
NS = 16
NW = NC * NS
LANES = 16

CHUNK = 128


def _combine_body(p_ref, s_ref, o_ref, hi_ref, lo_ref):
    comb = p_ref[...][:, None, :] + s_ref[...][None, :, :]
    o_ref[...] = comb
    hi = comb.astype(jnp.bfloat16)
    hi_ref[...] = hi
    lo_ref[...] = (comb - hi.astype(jnp.float32)).astype(jnp.bfloat16)


def _combine_tables(phoneme_table, stress_table):
    shape3 = (NUM_PHONEMES, NUM_STRESS, HIDDEN)
    out3, hi3, lo3 = pl.pallas_call(
        _combine_body,
        out_shape=[
            jax.ShapeDtypeStruct(shape3, jnp.float32),
            jax.ShapeDtypeStruct(shape3, jnp.bfloat16),
            jax.ShapeDtypeStruct(shape3, jnp.bfloat16),
        ],
    )(phoneme_table, stress_table)
    flat = (NUM_PHONEMES * NUM_STRESS, HIDDEN)
    return out3.reshape(flat), hi3.reshape(flat), lo3.reshape(flat)


def _make_sc_gather(n_tokens):
    assert n_tokens % (NW * CHUNK) == 0
    b_per_w = n_tokens // NW
    n_chunks = b_per_w // CHUNK
    assert n_chunks % 2 == 0 and n_chunks >= 4
    n_pairs = n_chunks // 2
    mesh = plsc.VectorSubcoreMesh(core_axis_name="c", subcore_axis_name="s")

    @functools.partial(
        pl.kernel,
        mesh=mesh,
        out_type=jax.ShapeDtypeStruct((n_tokens, HIDDEN), jnp.float32),
        scratch_types=[
            pltpu.VMEM((2, CHUNK), jnp.int32),
            pltpu.VMEM((2, CHUNK), jnp.int32),
            pltpu.VMEM((2, CHUNK), jnp.int32),
            pltpu.VMEM((2, CHUNK, HIDDEN), jnp.float32),
            pltpu.SemaphoreType.DMA,
            pltpu.SemaphoreType.DMA,
            pltpu.SemaphoreType.DMA,
            pltpu.SemaphoreType.DMA,
        ],
    )
    def sc_gather(tbl_hbm, pid_hbm, sid_hbm, out_hbm, pidv, sidv, idxv, rowsv,
                  g0, g1, o0, o1):
        gsem = (g0, g1)
        osem = (o0, o1)
        wid = lax.axis_index("s") * NC + lax.axis_index("c")
        base = wid * b_per_w

        def start_chunk(i, b):
            off = base + i * CHUNK
            pltpu.sync_copy(pid_hbm.at[pl.ds(off, CHUNK)], pidv.at[b])
            pltpu.sync_copy(sid_hbm.at[pl.ds(off, CHUNK)], sidv.at[b])
            for j in range(CHUNK // LANES):
                sl = pl.ds(j * LANES, LANES)
                idxv.at[b][sl] = pidv.at[b][sl] * NUM_STRESS + sidv.at[b][sl]
            pltpu.async_copy(tbl_hbm.at[idxv.at[b]], rowsv.at[b], gsem[b])

        def finish_chunk(i, b):
            off = base + i * CHUNK
            pltpu.make_async_copy(tbl_hbm.at[idxv.at[b]], rowsv.at[b],
                                  gsem[b]).wait()
            pltpu.async_copy(rowsv.at[b], out_hbm.at[pl.ds(off, CHUNK)], osem[b])

        def drain_out(i, b):
            pltpu.make_async_copy(rowsv.at[b],
                                  out_hbm.at[pl.ds(base + i * CHUNK, CHUNK)],
                                  osem[b]).wait()

        start_chunk(0, 0)
        start_chunk(1, 1)
        finish_chunk(0, 0)

        def pair_body(gg, carry):
            i0 = 2 * gg
            drain_out(i0 - 2, 0)
            start_chunk(i0, 0)
            finish_chunk(i0 - 1, 1)
            drain_out(i0 - 1, 1)
            start_chunk(i0 + 1, 1)
            finish_chunk(i0, 0)
            return carry

        lax.fori_loop(1, n_pairs, pair_body, 0)

        finish_chunk(n_chunks - 1, 1)
        drain_out(n_chunks - 2, 0)
        drain_out(n_chunks - 1, 1)

    return sc_gather


TC_BLOCK = 512


def _tc_gather_body(pid_ref, sid_ref, hi_ref, lo_ref, o_ref):
    fused = pid_ref[0, 0, :] * NUM_STRESS + sid_ref[0, 0, :]
    oh = (fused[:, None]
          == lax.broadcasted_iota(jnp.int32, (TC_BLOCK, NUM_PHONEMES * NUM_STRESS), 1))
    ohb = oh.astype(jnp.bfloat16)
    o_ref[...] = (
        jnp.dot(ohb, hi_ref[...], preferred_element_type=jnp.float32)
        + jnp.dot(ohb, lo_ref[...], preferred_element_type=jnp.float32))


def _tc_gather(tbl_hi, tbl_lo, pid, sid, n_tokens):
    n_blocks = n_tokens // TC_BLOCK
    return pl.pallas_call(
        _tc_gather_body,
        grid=(n_blocks,),
        in_specs=[
            pl.BlockSpec((1, 1, TC_BLOCK), lambda i: (i, 0, 0)),
            pl.BlockSpec((1, 1, TC_BLOCK), lambda i: (i, 0, 0)),
            pl.BlockSpec((NUM_PHONEMES * NUM_STRESS, HIDDEN), lambda i: (0, 0)),
            pl.BlockSpec((NUM_PHONEMES * NUM_STRESS, HIDDEN), lambda i: (0, 0)),
        ],
        out_specs=pl.BlockSpec((TC_BLOCK, HIDDEN), lambda i: (i, 0)),
        out_shape=jax.ShapeDtypeStruct((n_tokens, HIDDEN), jnp.float32),
        compiler_params=pltpu.CompilerParams(
            dimension_semantics=("arbitrary",)),
    )(pid.reshape(n_blocks, 1, TC_BLOCK), sid.reshape(n_blocks, 1, TC_BLOCK),
      tbl_hi, tbl_lo)


def kernel(phoneme_ids, stress_ids, phoneme_table, stress_table):
    B, L = phoneme_ids.shape
    n_tokens = B * L
    combined, tbl_hi, tbl_lo = _combine_tables(
        phoneme_table.astype(jnp.float32), stress_table.astype(jnp.float32))
    pid = phoneme_ids.reshape(-1).astype(jnp.int32)
    sid = stress_ids.reshape(-1).astype(jnp.int32)
    out = _tc_gather(tbl_hi, tbl_lo, pid, sid, n_tokens)
    return out.reshape(B, L, HIDDEN)


# speedup vs baseline: 2.0282x; 2.0282x over previous
import functools

import jax
import jax.numpy as jnp
from jax import lax
from jax.experimental import pallas as pl
from jax.experimental.pallas import tpu as pltpu
from jax.experimental.pallas import tpu_sc as plsc

NUM_PHONEMES = 80
NUM_STRESS = 4
HIDDEN = 256

NC = 2
NS = 16
NW = NC * NS
LANES = 16

CHUNK = 128


def _combine_body(p_ref, s_ref, o_ref, hi_ref, lo_ref):
    comb = p_ref[...][:, None, :] + s_ref[...][None, :, :]
    o_ref[...] = comb
    hi = comb.astype(jnp.bfloat16)
    hi_ref[...] = hi
    lo_ref[...] = (comb - hi.astype(jnp.float32)).astype(jnp.bfloat16)


def _combine_tables(phoneme_table, stress_table):
    shape3 = (NUM_PHONEMES, NUM_STRESS, HIDDEN)
    out3, hi3, lo3 = pl.pallas_call(
        _combine_body,
        out_shape=[
            jax.ShapeDtypeStruct(shape3, jnp.float32),
            jax.ShapeDtypeStruct(shape3, jnp.bfloat16),
            jax.ShapeDtypeStruct(shape3, jnp.bfloat16),
        ],
    )(phoneme_table, stress_table)
    flat = (NUM_PHONEMES * NUM_STRESS, HIDDEN)
    return out3.reshape(flat), hi3.reshape(flat), lo3.reshape(flat)


def _make_sc_gather(n_tokens):
    assert n_tokens % (NW * CHUNK) == 0
    b_per_w = n_tokens // NW
    n_chunks = b_per_w // CHUNK
    assert n_chunks % 2 == 0 and n_chunks >= 4
    n_pairs = n_chunks // 2
    mesh = plsc.VectorSubcoreMesh(core_axis_name="c", subcore_axis_name="s")

    @functools.partial(
        pl.kernel,
        mesh=mesh,
        out_type=jax.ShapeDtypeStruct((n_tokens, HIDDEN), jnp.float32),
        scratch_types=[
            pltpu.VMEM((2, CHUNK), jnp.int32),
            pltpu.VMEM((2, CHUNK), jnp.int32),
            pltpu.VMEM((2, CHUNK), jnp.int32),
            pltpu.VMEM((2, CHUNK, HIDDEN), jnp.float32),
            pltpu.SemaphoreType.DMA,
            pltpu.SemaphoreType.DMA,
            pltpu.SemaphoreType.DMA,
            pltpu.SemaphoreType.DMA,
        ],
    )
    def sc_gather(tbl_hbm, pid_hbm, sid_hbm, out_hbm, pidv, sidv, idxv, rowsv,
                  g0, g1, o0, o1):
        gsem = (g0, g1)
        osem = (o0, o1)
        wid = lax.axis_index("s") * NC + lax.axis_index("c")
        base = wid * b_per_w

        def start_chunk(i, b):
            off = base + i * CHUNK
            pltpu.sync_copy(pid_hbm.at[pl.ds(off, CHUNK)], pidv.at[b])
            pltpu.sync_copy(sid_hbm.at[pl.ds(off, CHUNK)], sidv.at[b])
            for j in range(CHUNK // LANES):
                sl = pl.ds(j * LANES, LANES)
                idxv.at[b][sl] = pidv.at[b][sl] * NUM_STRESS + sidv.at[b][sl]
            pltpu.async_copy(tbl_hbm.at[idxv.at[b]], rowsv.at[b], gsem[b])

        def finish_chunk(i, b):
            off = base + i * CHUNK
            pltpu.make_async_copy(tbl_hbm.at[idxv.at[b]], rowsv.at[b],
                                  gsem[b]).wait()
            pltpu.async_copy(rowsv.at[b], out_hbm.at[pl.ds(off, CHUNK)], osem[b])

        def drain_out(i, b):
            pltpu.make_async_copy(rowsv.at[b],
                                  out_hbm.at[pl.ds(base + i * CHUNK, CHUNK)],
                                  osem[b]).wait()

        start_chunk(0, 0)
        start_chunk(1, 1)
        finish_chunk(0, 0)

        def pair_body(gg, carry):
            i0 = 2 * gg
            drain_out(i0 - 2, 0)
            start_chunk(i0, 0)
            finish_chunk(i0 - 1, 1)
            drain_out(i0 - 1, 1)
            start_chunk(i0 + 1, 1)
            finish_chunk(i0, 0)
            return carry

        lax.fori_loop(1, n_pairs, pair_body, 0)

        finish_chunk(n_chunks - 1, 1)
        drain_out(n_chunks - 2, 0)
        drain_out(n_chunks - 1, 1)

    return sc_gather


TC_BLOCK = 2048


def _tc_gather_body(pid_ref, sid_ref, hi_ref, lo_ref, o_ref):
    fused = pid_ref[0, 0, :] * NUM_STRESS + sid_ref[0, 0, :]
    oh = (fused[:, None]
          == lax.broadcasted_iota(jnp.int32, (TC_BLOCK, NUM_PHONEMES * NUM_STRESS), 1))
    ohb = oh.astype(jnp.bfloat16)
    o_ref[...] = (
        jnp.dot(ohb, hi_ref[...], preferred_element_type=jnp.float32)
        + jnp.dot(ohb, lo_ref[...], preferred_element_type=jnp.float32))


def _tc_gather(tbl_hi, tbl_lo, pid, sid, n_tokens):
    n_blocks = n_tokens // TC_BLOCK
    return pl.pallas_call(
        _tc_gather_body,
        grid=(n_blocks,),
        in_specs=[
            pl.BlockSpec((1, 1, TC_BLOCK), lambda i: (i, 0, 0)),
            pl.BlockSpec((1, 1, TC_BLOCK), lambda i: (i, 0, 0)),
            pl.BlockSpec((NUM_PHONEMES * NUM_STRESS, HIDDEN), lambda i: (0, 0)),
            pl.BlockSpec((NUM_PHONEMES * NUM_STRESS, HIDDEN), lambda i: (0, 0)),
        ],
        out_specs=pl.BlockSpec((TC_BLOCK, HIDDEN), lambda i: (i, 0)),
        out_shape=jax.ShapeDtypeStruct((n_tokens, HIDDEN), jnp.float32),
        compiler_params=pltpu.CompilerParams(
            dimension_semantics=("arbitrary",)),
    )(pid.reshape(n_blocks, 1, TC_BLOCK), sid.reshape(n_blocks, 1, TC_BLOCK),
      tbl_hi, tbl_lo)


def kernel(phoneme_ids, stress_ids, phoneme_table, stress_table):
    B, L = phoneme_ids.shape
    n_tokens = B * L
    combined, tbl_hi, tbl_lo = _combine_tables(
        phoneme_table.astype(jnp.float32), stress_table.astype(jnp.float32))
    pid = phoneme_ids.reshape(-1).astype(jnp.int32)
    sid = stress_ids.reshape(-1).astype(jnp.int32)
    out = _tc_gather(tbl_hi, tbl_lo, pid, sid, n_tokens)
    return out.reshape(B, L, HIDDEN)


# speedup vs baseline: 2.2220x; 1.0955x over previous
import functools

import jax
import jax.numpy as jnp
from jax import lax
from jax.experimental import pallas as pl
from jax.experimental.pallas import tpu as pltpu
from jax.experimental.pallas import tpu_sc as plsc

NUM_PHONEMES = 80
NUM_STRESS = 4
HIDDEN = 256

NC = 2
NS = 16
NW = NC * NS
LANES = 16

CHUNK = 128


def _combine_body(p_ref, s_ref, o_ref, hi_ref, lo_ref):
    comb = p_ref[...][:, None, :] + s_ref[...][None, :, :]
    o_ref[...] = comb
    hi = comb.astype(jnp.bfloat16)
    hi_ref[...] = hi
    lo_ref[...] = (comb - hi.astype(jnp.float32)).astype(jnp.bfloat16)


def _combine_tables(phoneme_table, stress_table):
    shape3 = (NUM_PHONEMES, NUM_STRESS, HIDDEN)
    out3, hi3, lo3 = pl.pallas_call(
        _combine_body,
        out_shape=[
            jax.ShapeDtypeStruct(shape3, jnp.float32),
            jax.ShapeDtypeStruct(shape3, jnp.bfloat16),
            jax.ShapeDtypeStruct(shape3, jnp.bfloat16),
        ],
    )(phoneme_table, stress_table)
    flat = (NUM_PHONEMES * NUM_STRESS, HIDDEN)
    return out3.reshape(flat), hi3.reshape(flat), lo3.reshape(flat)


def _make_sc_gather(n_tokens):
    assert n_tokens % (NW * CHUNK) == 0
    b_per_w = n_tokens // NW
    n_chunks = b_per_w // CHUNK
    assert n_chunks % 2 == 0 and n_chunks >= 4
    n_pairs = n_chunks // 2
    mesh = plsc.VectorSubcoreMesh(core_axis_name="c", subcore_axis_name="s")

    @functools.partial(
        pl.kernel,
        mesh=mesh,
        out_type=jax.ShapeDtypeStruct((n_tokens, HIDDEN), jnp.float32),
        scratch_types=[
            pltpu.VMEM((2, CHUNK), jnp.int32),
            pltpu.VMEM((2, CHUNK), jnp.int32),
            pltpu.VMEM((2, CHUNK), jnp.int32),
            pltpu.VMEM((2, CHUNK, HIDDEN), jnp.float32),
            pltpu.SemaphoreType.DMA,
            pltpu.SemaphoreType.DMA,
            pltpu.SemaphoreType.DMA,
            pltpu.SemaphoreType.DMA,
        ],
    )
    def sc_gather(tbl_hbm, pid_hbm, sid_hbm, out_hbm, pidv, sidv, idxv, rowsv,
                  g0, g1, o0, o1):
        gsem = (g0, g1)
        osem = (o0, o1)
        wid = lax.axis_index("s") * NC + lax.axis_index("c")
        base = wid * b_per_w

        def start_chunk(i, b):
            off = base + i * CHUNK
            pltpu.sync_copy(pid_hbm.at[pl.ds(off, CHUNK)], pidv.at[b])
            pltpu.sync_copy(sid_hbm.at[pl.ds(off, CHUNK)], sidv.at[b])
            for j in range(CHUNK // LANES):
                sl = pl.ds(j * LANES, LANES)
                idxv.at[b][sl] = pidv.at[b][sl] * NUM_STRESS + sidv.at[b][sl]
            pltpu.async_copy(tbl_hbm.at[idxv.at[b]], rowsv.at[b], gsem[b])

        def finish_chunk(i, b):
            off = base + i * CHUNK
            pltpu.make_async_copy(tbl_hbm.at[idxv.at[b]], rowsv.at[b],
                                  gsem[b]).wait()
            pltpu.async_copy(rowsv.at[b], out_hbm.at[pl.ds(off, CHUNK)], osem[b])

        def drain_out(i, b):
            pltpu.make_async_copy(rowsv.at[b],
                                  out_hbm.at[pl.ds(base + i * CHUNK, CHUNK)],
                                  osem[b]).wait()

        start_chunk(0, 0)
        start_chunk(1, 1)
        finish_chunk(0, 0)

        def pair_body(gg, carry):
            i0 = 2 * gg
            drain_out(i0 - 2, 0)
            start_chunk(i0, 0)
            finish_chunk(i0 - 1, 1)
            drain_out(i0 - 1, 1)
            start_chunk(i0 + 1, 1)
            finish_chunk(i0, 0)
            return carry

        lax.fori_loop(1, n_pairs, pair_body, 0)

        finish_chunk(n_chunks - 1, 1)
        drain_out(n_chunks - 2, 0)
        drain_out(n_chunks - 1, 1)

    return sc_gather


TC_BLOCK = 4096


def _tc_gather_body(pid_ref, sid_ref, hi_ref, lo_ref, o_ref):
    fused = pid_ref[0, 0, :] * NUM_STRESS + sid_ref[0, 0, :]
    oh = (fused[:, None]
          == lax.broadcasted_iota(jnp.int32, (TC_BLOCK, NUM_PHONEMES * NUM_STRESS), 1))
    ohb = oh.astype(jnp.bfloat16)
    o_ref[...] = (
        jnp.dot(ohb, hi_ref[...], preferred_element_type=jnp.float32)
        + jnp.dot(ohb, lo_ref[...], preferred_element_type=jnp.float32))


def _tc_gather(tbl_hi, tbl_lo, pid, sid, n_tokens):
    n_blocks = n_tokens // TC_BLOCK
    return pl.pallas_call(
        _tc_gather_body,
        grid=(n_blocks,),
        in_specs=[
            pl.BlockSpec((1, 1, TC_BLOCK), lambda i: (i, 0, 0)),
            pl.BlockSpec((1, 1, TC_BLOCK), lambda i: (i, 0, 0)),
            pl.BlockSpec((NUM_PHONEMES * NUM_STRESS, HIDDEN), lambda i: (0, 0)),
            pl.BlockSpec((NUM_PHONEMES * NUM_STRESS, HIDDEN), lambda i: (0, 0)),
        ],
        out_specs=pl.BlockSpec((TC_BLOCK, HIDDEN), lambda i: (i, 0)),
        out_shape=jax.ShapeDtypeStruct((n_tokens, HIDDEN), jnp.float32),
        compiler_params=pltpu.CompilerParams(
            dimension_semantics=("arbitrary",)),
    )(pid.reshape(n_blocks, 1, TC_BLOCK), sid.reshape(n_blocks, 1, TC_BLOCK),
      tbl_hi, tbl_lo)


def kernel(phoneme_ids, stress_ids, phoneme_table, stress_table):
    B, L = phoneme_ids.shape
    n_tokens = B * L
    combined, tbl_hi, tbl_lo = _combine_tables(
        phoneme_table.astype(jnp.float32), stress_table.astype(jnp.float32))
    pid = phoneme_ids.reshape(-1).astype(jnp.int32)
    sid = stress_ids.reshape(-1).astype(jnp.int32)
    out = _tc_gather(tbl_hi, tbl_lo, pid, sid, n_tokens)
    return out.reshape(B, L, HIDDEN)


# speedup vs baseline: 2.3189x; 1.0436x over previous
import functools

import jax
import jax.numpy as jnp
from jax import lax
from jax.experimental import pallas as pl
from jax.experimental.pallas import tpu as pltpu
from jax.experimental.pallas import tpu_sc as plsc

NUM_PHONEMES = 80
NUM_STRESS = 4
HIDDEN = 256

NC = 2
NS = 16
NW = NC * NS
LANES = 16

CHUNK = 128


def _combine_body(p_ref, s_ref, o_ref, hi_ref, lo_ref):
    comb = p_ref[...][:, None, :] + s_ref[...][None, :, :]
    o_ref[...] = comb
    hi = comb.astype(jnp.bfloat16)
    hi_ref[...] = hi
    lo_ref[...] = (comb - hi.astype(jnp.float32)).astype(jnp.bfloat16)


def _combine_tables(phoneme_table, stress_table):
    shape3 = (NUM_PHONEMES, NUM_STRESS, HIDDEN)
    out3, hi3, lo3 = pl.pallas_call(
        _combine_body,
        out_shape=[
            jax.ShapeDtypeStruct(shape3, jnp.float32),
            jax.ShapeDtypeStruct(shape3, jnp.bfloat16),
            jax.ShapeDtypeStruct(shape3, jnp.bfloat16),
        ],
    )(phoneme_table, stress_table)
    flat = (NUM_PHONEMES * NUM_STRESS, HIDDEN)
    return out3.reshape(flat), hi3.reshape(flat), lo3.reshape(flat)


def _make_sc_gather(n_tokens):
    assert n_tokens % (NW * CHUNK) == 0
    b_per_w = n_tokens // NW
    n_chunks = b_per_w // CHUNK
    assert n_chunks % 2 == 0 and n_chunks >= 4
    n_pairs = n_chunks // 2
    mesh = plsc.VectorSubcoreMesh(core_axis_name="c", subcore_axis_name="s")

    @functools.partial(
        pl.kernel,
        mesh=mesh,
        out_type=jax.ShapeDtypeStruct((n_tokens, HIDDEN), jnp.float32),
        scratch_types=[
            pltpu.VMEM((2, CHUNK), jnp.int32),
            pltpu.VMEM((2, CHUNK), jnp.int32),
            pltpu.VMEM((2, CHUNK), jnp.int32),
            pltpu.VMEM((2, CHUNK, HIDDEN), jnp.float32),
            pltpu.SemaphoreType.DMA,
            pltpu.SemaphoreType.DMA,
            pltpu.SemaphoreType.DMA,
            pltpu.SemaphoreType.DMA,
        ],
    )
    def sc_gather(tbl_hbm, pid_hbm, sid_hbm, out_hbm, pidv, sidv, idxv, rowsv,
                  g0, g1, o0, o1):
        gsem = (g0, g1)
        osem = (o0, o1)
        wid = lax.axis_index("s") * NC + lax.axis_index("c")
        base = wid * b_per_w

        def start_chunk(i, b):
            off = base + i * CHUNK
            pltpu.sync_copy(pid_hbm.at[pl.ds(off, CHUNK)], pidv.at[b])
            pltpu.sync_copy(sid_hbm.at[pl.ds(off, CHUNK)], sidv.at[b])
            for j in range(CHUNK // LANES):
                sl = pl.ds(j * LANES, LANES)
                idxv.at[b][sl] = pidv.at[b][sl] * NUM_STRESS + sidv.at[b][sl]
            pltpu.async_copy(tbl_hbm.at[idxv.at[b]], rowsv.at[b], gsem[b])

        def finish_chunk(i, b):
            off = base + i * CHUNK
            pltpu.make_async_copy(tbl_hbm.at[idxv.at[b]], rowsv.at[b],
                                  gsem[b]).wait()
            pltpu.async_copy(rowsv.at[b], out_hbm.at[pl.ds(off, CHUNK)], osem[b])

        def drain_out(i, b):
            pltpu.make_async_copy(rowsv.at[b],
                                  out_hbm.at[pl.ds(base + i * CHUNK, CHUNK)],
                                  osem[b]).wait()

        start_chunk(0, 0)
        start_chunk(1, 1)
        finish_chunk(0, 0)

        def pair_body(gg, carry):
            i0 = 2 * gg
            drain_out(i0 - 2, 0)
            start_chunk(i0, 0)
            finish_chunk(i0 - 1, 1)
            drain_out(i0 - 1, 1)
            start_chunk(i0 + 1, 1)
            finish_chunk(i0, 0)
            return carry

        lax.fori_loop(1, n_pairs, pair_body, 0)

        finish_chunk(n_chunks - 1, 1)
        drain_out(n_chunks - 2, 0)
        drain_out(n_chunks - 1, 1)

    return sc_gather


TC_BLOCK = 8192


def _tc_gather_body(pid_ref, sid_ref, hi_ref, lo_ref, o_ref):
    fused = pid_ref[0, 0, :] * NUM_STRESS + sid_ref[0, 0, :]
    oh = (fused[:, None]
          == lax.broadcasted_iota(jnp.int32, (TC_BLOCK, NUM_PHONEMES * NUM_STRESS), 1))
    ohb = oh.astype(jnp.bfloat16)
    o_ref[...] = (
        jnp.dot(ohb, hi_ref[...], preferred_element_type=jnp.float32)
        + jnp.dot(ohb, lo_ref[...], preferred_element_type=jnp.float32))


def _tc_gather(tbl_hi, tbl_lo, pid, sid, n_tokens):
    n_blocks = n_tokens // TC_BLOCK
    return pl.pallas_call(
        _tc_gather_body,
        grid=(n_blocks,),
        in_specs=[
            pl.BlockSpec((1, 1, TC_BLOCK), lambda i: (i, 0, 0)),
            pl.BlockSpec((1, 1, TC_BLOCK), lambda i: (i, 0, 0)),
            pl.BlockSpec((NUM_PHONEMES * NUM_STRESS, HIDDEN), lambda i: (0, 0)),
            pl.BlockSpec((NUM_PHONEMES * NUM_STRESS, HIDDEN), lambda i: (0, 0)),
        ],
        out_specs=pl.BlockSpec((TC_BLOCK, HIDDEN), lambda i: (i, 0)),
        out_shape=jax.ShapeDtypeStruct((n_tokens, HIDDEN), jnp.float32),
        compiler_params=pltpu.CompilerParams(
            dimension_semantics=("arbitrary",)),
    )(pid.reshape(n_blocks, 1, TC_BLOCK), sid.reshape(n_blocks, 1, TC_BLOCK),
      tbl_hi, tbl_lo)


def kernel(phoneme_ids, stress_ids, phoneme_table, stress_table):
    B, L = phoneme_ids.shape
    n_tokens = B * L
    combined, tbl_hi, tbl_lo = _combine_tables(
        phoneme_table.astype(jnp.float32), stress_table.astype(jnp.float32))
    pid = phoneme_ids.reshape(-1).astype(jnp.int32)
    sid = stress_ids.reshape(-1).astype(jnp.int32)
    out = _tc_gather(tbl_hi, tbl_lo, pid, sid, n_tokens)
    return out.reshape(B, L, HIDDEN)
